# baseline (speedup 1.0000x reference)
import functools

import jax
import jax.numpy as jnp
from jax.experimental import pallas as pl
from jax.experimental.pallas import tpu as pltpu

_MAX_SPAN_LEN = 16
_TPS = 3


def _pool_body(tl_sm, nt_sm, sn_sm, emb_hbm, so_ref, cv_ref,
               out_ref, mask_ref, gbuf, sem, *, S, D, TMAX):
    b = pl.program_id(0)
    nb = pl.num_programs(0)
    slot = jax.lax.rem(b, 2)
    nslot = 1 - slot

    def issue(bb, sl):
        nt = nt_sm[bb]

        def one(j, carry):
            row = pl.multiple_of(tl_sm[bb * TMAX + j], 8)
            dst = pl.multiple_of(j * 8, 8)
            pltpu.make_async_copy(
                emb_hbm.at[bb, pl.ds(row, 8), :],
                gbuf.at[sl, pl.ds(dst, 8), :],
                sem.at[sl],
            ).start()
            return carry

        jax.lax.fori_loop(0, nt, one, 0)

    @pl.when(b == 0)
    def _():
        gbuf[...] = jnp.zeros_like(gbuf)
        issue(b, slot)

    @pl.when(b + 1 < nb)
    def _():
        issue(b + 1, nslot)

    sn = sn_sm[b]
    span_iota = jax.lax.broadcasted_iota(jnp.int32, (1, S), 1)
    valid_span = span_iota < sn
    mask_ref[0] = valid_span.astype(jnp.int32)

    so = so_ref[0]
    cv = cv_ref[0]
    inv = 1.0 / jnp.maximum(cv, 1).astype(jnp.float32)
    scale = jnp.where(valid_span & (cv > 0), inv, 0.0)
    k_iota = jax.lax.broadcasted_iota(jnp.int32, (S * _TPS * 8, S), 0)
    wt = jnp.where((k_iota >= so) & (k_iota < so + cv), scale, 0.0)

    nt_b = nt_sm[b]
    pltpu.make_async_copy(
        emb_hbm.at[b, pl.ds(0, 8 * nt_b), :],
        gbuf.at[slot, pl.ds(0, 8 * nt_b), :],
        sem.at[slot],
    ).wait()

    out_ref[0] = jax.lax.dot_general(
        wt, gbuf[slot], (((0,), (0,)), ((), ())),
        preferred_element_type=jnp.float32)


def kernel(embeddings, span_positions, span_nums, repeated_vector):
    B, L, D = embeddings.shape
    S = span_positions.shape[1]
    T = L // 8
    TMAX = min(_TPS * S, T)

    sp = span_positions.astype(jnp.int32) + 1
    starts = sp[..., 0]
    ends = sp[..., 1]
    lo = starts & 7
    cnt = jnp.clip(ends - starts, 0, _MAX_SPAN_LEN)
    t0 = starts >> 3
    ext = lo + jnp.maximum(cnt, 1)

    k3 = jnp.arange(_TPS, dtype=jnp.int32)
    span_t = t0[..., None] + k3
    need = (k3 * 8) < ext[..., None]

    tid = jnp.arange(T, dtype=jnp.int32)
    present = jnp.any((span_t[..., None] == tid) & need[..., None],
                      axis=(1, 2))
    cum = jnp.cumsum(present.astype(jnp.int32), axis=1)
    n_tiles = cum[:, -1]
    pos0 = jnp.take_along_axis(cum, t0, axis=1) - 1
    slab_off = pos0 * 8 + lo

    sort_key = jnp.where(present, tid[None, :], T + tid[None, :])
    tile_rows = (jnp.sort(sort_key, axis=1)[:, :TMAX] % T) * 8

    sn = span_nums.astype(jnp.int32)
    body = functools.partial(_pool_body, S=S, D=D, TMAX=TMAX)
    grid_spec = pltpu.PrefetchScalarGridSpec(
        num_scalar_prefetch=3,
        grid=(B,),
        in_specs=[pl.BlockSpec(memory_space=pl.ANY),
                  pl.BlockSpec((1, 1, S), lambda b, *_: (b, 0, 0)),
                  pl.BlockSpec((1, 1, S), lambda b, *_: (b, 0, 0))],
        out_specs=[pl.BlockSpec((1, S, D), lambda b, *_: (b, 0, 0)),
                   pl.BlockSpec((1, 1, S), lambda b, *_: (b, 0, 0))],
        scratch_shapes=[
            pltpu.VMEM((2, S * _TPS * 8, D), jnp.float32),
            pltpu.SemaphoreType.DMA((2,)),
        ],
    )
    pooled, maski = pl.pallas_call(
        body,
        grid_spec=grid_spec,
        out_shape=[jax.ShapeDtypeStruct((B, S, D), jnp.float32),
                   jax.ShapeDtypeStruct((B, 1, S), jnp.int32)],
        compiler_params=pltpu.CompilerParams(
            dimension_semantics=("arbitrary",),
        ),
        name="span_mean_pool_tile_gather",
    )(tile_rows.reshape(-1), n_tiles, sn, embeddings,
      slab_off.reshape(B, 1, S), cnt.reshape(B, 1, S))
    return pooled, maski.reshape(B, S) > 0


# speedup vs baseline: 1.2312x; 1.2312x over previous
import functools

import jax
import jax.numpy as jnp
from jax.experimental import pallas as pl
from jax.experimental.pallas import tpu as pltpu

_MAX_SPAN_LEN = 16
_TPS = 3


def _pool_body(tl_sm, nt_sm, sn_sm, emb_hbm, so_ref, cv_ref,
               out_ref, mask_ref, gbuf, sem, *, S, D, TMAX):
    b = pl.program_id(0)
    nb = pl.num_programs(0)
    slot = jax.lax.rem(b, 2)
    nslot = 1 - slot

    def issue(bb, sl):
        nt = nt_sm[bb]

        def one(j, carry):
            row = pl.multiple_of(tl_sm[bb * TMAX + j], 8)
            dst = pl.multiple_of(j * 8, 8)
            pltpu.make_async_copy(
                emb_hbm.at[bb, pl.ds(row, 8), :],
                gbuf.at[sl, pl.ds(dst, 8), :],
                sem.at[sl],
            ).start()
            return carry

        jax.lax.fori_loop(0, nt, one, 0)

    @pl.when(b == 0)
    def _():
        gbuf[...] = jnp.zeros_like(gbuf)
        issue(b, slot)

    @pl.when(b + 1 < nb)
    def _():
        issue(b + 1, nslot)

    sn = sn_sm[b]
    span_iota = jax.lax.broadcasted_iota(jnp.int32, (1, S), 1)
    valid_span = span_iota < sn
    mask_ref[0] = valid_span.astype(jnp.int32)

    so = so_ref[0]
    cv = cv_ref[0]
    inv = 1.0 / jnp.maximum(cv, 1).astype(jnp.float32)
    scale = jnp.where(valid_span & (cv > 0), inv, 0.0)
    k_iota = jax.lax.broadcasted_iota(jnp.int32, (S * _TPS * 8, S), 0)
    wt = jnp.where((k_iota >= so) & (k_iota < so + cv), scale, 0.0)

    nt_b = nt_sm[b]
    pltpu.make_async_copy(
        emb_hbm.at[b, pl.ds(0, 8 * nt_b), :],
        gbuf.at[slot, pl.ds(0, 8 * nt_b), :],
        sem.at[slot],
    ).wait()

    out_ref[0] = jax.lax.dot_general(
        wt, gbuf[slot], (((0,), (0,)), ((), ())),
        preferred_element_type=jnp.float32)


def kernel(embeddings, span_positions, span_nums, repeated_vector):
    B, L, D = embeddings.shape
    S = span_positions.shape[1]
    T = L // 8
    TMAX = min(_TPS * S, T)

    sp = span_positions.astype(jnp.int32) + 1
    starts = sp[..., 0]
    ends = sp[..., 1]
    lo = starts & 7
    cnt = jnp.clip(ends - starts, 0, _MAX_SPAN_LEN)
    t0 = starts >> 3
    ext = lo + jnp.maximum(cnt, 1)

    ntile = (ext + 7) >> 3
    cpos = jnp.cumsum(ntile, axis=1)
    pos = cpos - ntile
    n_tiles = cpos[:, -1]
    slab_off = pos * 8 + lo

    j_iota = jnp.arange(TMAX, dtype=jnp.int32)[None, :, None]
    posb = pos[:, None, :]
    sel = (j_iota >= posb) & (j_iota < posb + ntile[:, None, :])
    contrib = t0[:, None, :] + (j_iota - posb)
    tile_rows = jnp.sum(jnp.where(sel, contrib, 0), axis=2) * 8

    sn = span_nums.astype(jnp.int32)
    body = functools.partial(_pool_body, S=S, D=D, TMAX=TMAX)
    grid_spec = pltpu.PrefetchScalarGridSpec(
        num_scalar_prefetch=3,
        grid=(B,),
        in_specs=[pl.BlockSpec(memory_space=pl.ANY),
                  pl.BlockSpec((1, 1, S), lambda b, *_: (b, 0, 0)),
                  pl.BlockSpec((1, 1, S), lambda b, *_: (b, 0, 0))],
        out_specs=[pl.BlockSpec((1, S, D), lambda b, *_: (b, 0, 0)),
                   pl.BlockSpec((1, 1, S), lambda b, *_: (b, 0, 0))],
        scratch_shapes=[
            pltpu.VMEM((2, S * _TPS * 8, D), jnp.float32),
            pltpu.SemaphoreType.DMA((2,)),
        ],
    )
    pooled, maski = pl.pallas_call(
        body,
        grid_spec=grid_spec,
        out_shape=[jax.ShapeDtypeStruct((B, S, D), jnp.float32),
                   jax.ShapeDtypeStruct((B, 1, S), jnp.int32)],
        compiler_params=pltpu.CompilerParams(
            dimension_semantics=("arbitrary",),
        ),
        name="span_mean_pool_tile_gather",
    )(tile_rows.reshape(-1), n_tiles, sn, embeddings,
      slab_off.reshape(B, 1, S), cnt.reshape(B, 1, S))
    return pooled, maski.reshape(B, S) > 0


# speedup vs baseline: 1.4646x; 1.1895x over previous
import functools

import jax
import jax.numpy as jnp
from jax.experimental import pallas as pl
from jax.experimental.pallas import tpu as pltpu

_MAX_SPAN_LEN = 16
_WIN = 24


def _pool_body(pk_sm, nsp_sm, sn_sm, emb_hbm, sv_ref, ev_ref,
               out_ref, mask_ref, gbuf, sem16, sem8, *, S, D):
    b = pl.program_id(0)
    nb = pl.num_programs(0)
    slot = jax.lax.rem(b, 2)
    nslot = 1 - slot

    def issue(bb, sl):
        for mi in range(S):
            v = pk_sm[bb * S + mi]
            base = pl.multiple_of(v & 0xFFFF, 8)
            pltpu.make_async_copy(
                emb_hbm.at[bb, pl.ds(base, 16), :],
                gbuf.at[sl, pl.ds(mi * _WIN, 16), :],
                sem16.at[sl],
            ).start()

            @pl.when((v >> 16) != 0)
            def _():
                pltpu.make_async_copy(
                    emb_hbm.at[bb, pl.ds(base + 16, 8), :],
                    gbuf.at[sl, pl.ds(mi * _WIN + 16, 8), :],
                    sem8.at[sl],
                ).start()

    @pl.when(b == 0)
    def _():
        issue(b, slot)

    @pl.when(b + 1 < nb)
    def _():
        issue(b + 1, nslot)

    sn = sn_sm[b]
    span_iota = jax.lax.broadcasted_iota(jnp.int32, (1, S), 1)
    valid_span = span_iota < sn
    mask_ref[0] = valid_span.astype(jnp.int32)

    sv = sv_ref[0]
    ev = ev_ref[0]
    lo = sv - ((sv >> 3) << 3)
    cnt = jnp.minimum(ev - sv, _MAX_SPAN_LEN)
    hi = lo + cnt
    inv = 1.0 / jnp.maximum(cnt, 1).astype(jnp.float32)
    scale = jnp.where(valid_span & (cnt > 0), inv, 0.0)

    k_iota = jax.lax.broadcasted_iota(jnp.int32, (S * _WIN, S), 0)
    mi_iota = jax.lax.broadcasted_iota(jnp.int32, (S * _WIN, S), 1)
    off = k_iota - mi_iota * _WIN
    wt = jnp.where((off >= lo) & (off < hi), scale, 0.0)

    pltpu.make_async_copy(
        emb_hbm.at[b, pl.ds(0, S * 16), :],
        gbuf.at[slot, pl.ds(0, S * 16), :],
        sem16.at[slot],
    ).wait()
    ns = nsp_sm[b]

    @pl.when(ns > 0)
    def _():
        pltpu.make_async_copy(
            emb_hbm.at[b, pl.ds(0, 8 * ns), :],
            gbuf.at[slot, pl.ds(0, 8 * ns), :],
            sem8.at[slot],
        ).wait()

    out_ref[0] = jax.lax.dot_general(
        wt, gbuf[slot], (((0,), (0,)), ((), ())),
        preferred_element_type=jnp.float32)


def kernel(embeddings, span_positions, span_nums, repeated_vector):
    B, L, D = embeddings.shape
    S = span_positions.shape[1]
    sp = span_positions.astype(jnp.int32) + 1
    starts = sp[..., 0]
    ends = sp[..., 1]
    lo = starts & 7
    cnt = jnp.clip(ends - starts, 0, _MAX_SPAN_LEN)
    base_rows = (starts >> 3) * 8
    spill = (lo + jnp.maximum(cnt, 1)) > 16
    spill = spill | (jnp.arange(B, dtype=jnp.int32)[:, None] <= 1)
    pk = (base_rows | (spill.astype(jnp.int32) << 16)).reshape(-1)
    n_spill = spill.astype(jnp.int32).sum(axis=1)
    sn = span_nums.astype(jnp.int32)
    sv = starts.reshape(B, 1, S)
    ev = ends.reshape(B, 1, S)

    body = functools.partial(_pool_body, S=S, D=D)
    grid_spec = pltpu.PrefetchScalarGridSpec(
        num_scalar_prefetch=3,
        grid=(B,),
        in_specs=[pl.BlockSpec(memory_space=pl.ANY),
                  pl.BlockSpec((1, 1, S), lambda b, *_: (b, 0, 0)),
                  pl.BlockSpec((1, 1, S), lambda b, *_: (b, 0, 0))],
        out_specs=[pl.BlockSpec((1, S, D), lambda b, *_: (b, 0, 0)),
                   pl.BlockSpec((1, 1, S), lambda b, *_: (b, 0, 0))],
        scratch_shapes=[
            pltpu.VMEM((2, S * _WIN, D), jnp.float32),
            pltpu.SemaphoreType.DMA((2,)),
            pltpu.SemaphoreType.DMA((2,)),
        ],
    )
    pooled, maski = pl.pallas_call(
        body,
        grid_spec=grid_spec,
        out_shape=[jax.ShapeDtypeStruct((B, S, D), jnp.float32),
                   jax.ShapeDtypeStruct((B, 1, S), jnp.int32)],
        compiler_params=pltpu.CompilerParams(
            dimension_semantics=("arbitrary",),
        ),
        name="span_mean_pool_dma_mxu",
    )(pk, n_spill, sn, embeddings, sv, ev)
    return pooled, maski.reshape(B, S) > 0


# speedup vs baseline: 1.9302x; 1.3179x over previous
import functools

import jax
import jax.numpy as jnp
from jax.experimental import pallas as pl
from jax.experimental.pallas import tpu as pltpu

_MAX_SPAN_LEN = 16
_WIN = 24


def _pool_body(pk_sm, nsp_sm, sn_sm, emb_hbm, sv_ref, ev_ref,
               out_ref, mask_ref, gbuf, sem16, sem8, *, S, D):
    b = pl.program_id(0)
    nb = pl.num_programs(0)
    slot = jax.lax.rem(b, 4)

    def issue(bb, sl):
        for mi in range(S):
            v = pk_sm[bb * S + mi]
            base = pl.multiple_of(v & 0xFFFF, 8)
            pltpu.make_async_copy(
                emb_hbm.at[bb, pl.ds(base, 16), :],
                gbuf.at[sl, pl.ds(mi * _WIN, 16), :],
                sem16.at[sl],
            ).start()

            @pl.when((v >> 16) != 0)
            def _():
                pltpu.make_async_copy(
                    emb_hbm.at[bb, pl.ds(base + 16, 8), :],
                    gbuf.at[sl, pl.ds(mi * _WIN + 16, 8), :],
                    sem8.at[sl],
                ).start()

    @pl.when(b == 0)
    def _():
        issue(0, 0)
        issue(1, 1)

    @pl.when(b + 2 < nb)
    def _():
        issue(b + 2, jax.lax.rem(b + 2, 4))

    sn = sn_sm[b]
    span_iota = jax.lax.broadcasted_iota(jnp.int32, (1, S), 1)
    valid_span = span_iota < sn
    mask_ref[0] = valid_span.astype(jnp.int32)

    sv = sv_ref[0]
    ev = ev_ref[0]
    lo = sv - ((sv >> 3) << 3)
    cnt = jnp.minimum(ev - sv, _MAX_SPAN_LEN)
    hi = lo + cnt
    inv = 1.0 / jnp.maximum(cnt, 1).astype(jnp.float32)
    scale = jnp.where(valid_span & (cnt > 0), inv, 0.0)

    k_iota = jax.lax.broadcasted_iota(jnp.int32, (S * _WIN, S), 0)
    mi_iota = jax.lax.broadcasted_iota(jnp.int32, (S * _WIN, S), 1)
    off = k_iota - mi_iota * _WIN
    wt = jnp.where((off >= lo) & (off < hi), scale, 0.0)

    pltpu.make_async_copy(
        emb_hbm.at[b, pl.ds(0, S * 16), :],
        gbuf.at[slot, pl.ds(0, S * 16), :],
        sem16.at[slot],
    ).wait()
    ns = nsp_sm[b]

    @pl.when(ns > 0)
    def _():
        pltpu.make_async_copy(
            emb_hbm.at[b, pl.ds(0, 8 * ns), :],
            gbuf.at[slot, pl.ds(0, 8 * ns), :],
            sem8.at[slot],
        ).wait()

    out_ref[0] = jax.lax.dot_general(
        wt, gbuf[slot], (((0,), (0,)), ((), ())),
        preferred_element_type=jnp.float32)


def kernel(embeddings, span_positions, span_nums, repeated_vector):
    B, L, D = embeddings.shape
    S = span_positions.shape[1]
    sp = span_positions.astype(jnp.int32) + 1
    starts = sp[..., 0]
    ends = sp[..., 1]
    lo = starts & 7
    cnt = jnp.clip(ends - starts, 0, _MAX_SPAN_LEN)
    base_rows = (starts >> 3) * 8
    spill = (lo + jnp.maximum(cnt, 1)) > 16
    spill = spill | (jnp.arange(B, dtype=jnp.int32)[:, None] <= 3)
    pk = (base_rows | (spill.astype(jnp.int32) << 16)).reshape(-1)
    n_spill = spill.astype(jnp.int32).sum(axis=1)
    sn = span_nums.astype(jnp.int32)
    sv = starts.reshape(B, 1, S)
    ev = ends.reshape(B, 1, S)

    body = functools.partial(_pool_body, S=S, D=D)
    grid_spec = pltpu.PrefetchScalarGridSpec(
        num_scalar_prefetch=3,
        grid=(B,),
        in_specs=[pl.BlockSpec(memory_space=pl.ANY),
                  pl.BlockSpec((1, 1, S), lambda b, *_: (b, 0, 0)),
                  pl.BlockSpec((1, 1, S), lambda b, *_: (b, 0, 0))],
        out_specs=[pl.BlockSpec((1, S, D), lambda b, *_: (b, 0, 0)),
                   pl.BlockSpec((1, 1, S), lambda b, *_: (b, 0, 0))],
        scratch_shapes=[
            pltpu.VMEM((4, S * _WIN, D), jnp.float32),
            pltpu.SemaphoreType.DMA((4,)),
            pltpu.SemaphoreType.DMA((4,)),
        ],
    )
    pooled, maski = pl.pallas_call(
        body,
        grid_spec=grid_spec,
        out_shape=[jax.ShapeDtypeStruct((B, S, D), jnp.float32),
                   jax.ShapeDtypeStruct((B, 1, S), jnp.int32)],
        compiler_params=pltpu.CompilerParams(
            dimension_semantics=("arbitrary",),
        ),
        name="span_mean_pool_dma_mxu",
    )(pk, n_spill, sn, embeddings, sv, ev)
    return pooled, maski.reshape(B, S) > 0
